# baseline (speedup 1.0000x reference)
import jax
import jax.numpy as jnp
from jax import lax
from jax.experimental import pallas as pl
from jax.experimental.pallas import tpu as pltpu
from jax.experimental.pallas import tpu_sc as plsc

D0 = 4096
SUMD = 2 * D0
F = 256
H = 128
NCAND = 16
BQ = 32768
L = 16
NC, NS = 2, 16
NW = NC * NS
K = 64
RB = 1024


def _prep_body(feats_ref, cand_ref, values_ref, scale_ref, bias_ref,
               m_ref, aux_ref, params_ref, w3t_ref):
    i = pl.program_id(0)
    is0 = i < (pl.num_programs(0) // 2)
    f = feats_ref[...]
    cand = cand_ref[0]
    vals = values_ref[0]
    c0v = cand[:, 0]
    best = jnp.abs(f - c0v)
    q = jnp.broadcast_to(c0v, f.shape)
    for k in range(1, NCAND):
        ck = cand[:, k]
        d = jnp.abs(f - ck)
        better = d < best
        q = jnp.where(better, ck, q)
        best = jnp.where(better, d, best)
    w1 = vals[1] - vals[3]
    w2 = vals[2] - vals[3]
    w3 = vals[0] - vals[1] - vals[2] + vals[3]
    wa = jnp.where(is0, w1, w2)
    a0 = jnp.sum(q[:, :H] * wa[:H], axis=1)
    a1 = jnp.sum(q[:, H:] * wa[H:], axis=1)
    u = lax.bitcast_convert_type(q, jnp.uint32)
    bits = (u + jnp.uint32(0x7FFF) + ((u >> 16) & jnp.uint32(1))) >> 16
    m_ref[...] = lax.bitcast_convert_type(
        bits[:, :H] | (bits[:, H:] << 16), jnp.int32)
    aux_ref[...] = jnp.stack([a0, a1], axis=0)
    w3t_ref[...] = jnp.stack([w3[:H], w3[H:]], axis=0)
    base = vals[3]
    c0 = jnp.sum(base[:H])
    c1 = jnp.sum(base[H:])
    esc = jnp.exp(scale_ref[0])
    b = bias_ref[0]
    params_ref[...] = jnp.concatenate([
        jnp.full((1, L), esc, jnp.float32),
        jnp.full((1, L), b, jnp.float32),
        jnp.full((1, L), c0, jnp.float32),
        jnp.full((1, L), c1, jnp.float32),
    ], axis=0)


def _prep(feats, candidates, values, scale, bias):
    grid = SUMD // RB
    return pl.pallas_call(
        _prep_body,
        grid=(grid,),
        in_specs=[
            pl.BlockSpec((RB, F), lambda i: (i, 0)),
            pl.BlockSpec((1, F, NCAND), lambda i: (i // (SUMD // RB // 2), 0, 0)),
            pl.BlockSpec((1, 4, F), lambda i: (0, 0, 0)),
            pl.BlockSpec(memory_space=pltpu.SMEM),
            pl.BlockSpec(memory_space=pltpu.SMEM),
        ],
        out_specs=[
            pl.BlockSpec((RB, H), lambda i: (i, 0)),
            pl.BlockSpec((2, RB), lambda i: (0, i)),
            pl.BlockSpec((4, L), lambda i: (0, 0)),
            pl.BlockSpec((2, H), lambda i: (0, 0)),
        ],
        out_shape=[
            jax.ShapeDtypeStruct((SUMD, H), jnp.int32),
            jax.ShapeDtypeStruct((2, SUMD), jnp.float32),
            jax.ShapeDtypeStruct((4, L), jnp.float32),
            jax.ShapeDtypeStruct((2, H), jnp.float32),
        ],
    )(feats, candidates, values, scale, bias)


def _sc_body(m_hbm, aux_hbm, i0_hbm, i1_hbm, params_hbm, w3t_hbm, out_hbm,
             aux0_v, aux1_v, params_v, w3_v,
             iall0_v, iall1_v,
             ra0_v, ra1_v, rb0_v, rb1_v, rc0_v, rc1_v, rd0_v, rd1_v,
             s0_v, p0_v, p1_v,
             semA0, semA1, semB0, semB1, semC0, semC1, semD0, semD1):
    wid = lax.axis_index("s") * NC + lax.axis_index("c")
    per_w = BQ // NW
    nchunk = per_w // K
    w_base = wid * per_w
    lanes = lax.broadcasted_iota(jnp.int32, (L,), 0)
    last = nchunk - 1

    pltpu.sync_copy(aux_hbm.at[0], aux0_v)
    pltpu.sync_copy(aux_hbm.at[1], aux1_v)
    pltpu.sync_copy(params_hbm, params_v)
    pltpu.sync_copy(w3t_hbm, w3_v)
    escv = params_v[0, :]
    biasv = params_v[1, :]
    c0v = params_v[2, :]
    c1v = params_v[3, :]
    wev = [w3_v[0, pl.ds(u * L, L)] for u in range(8)]
    wov = [w3_v[1, pl.ds(u * L, L)] for u in range(8)]

    pltpu.sync_copy(i0_hbm.at[pl.ds(w_base, per_w)], iall0_v)
    pltpu.sync_copy(i1_hbm.at[pl.ds(w_base, per_w)], iall1_v)

    def fire(ci, r0_v, r1_v, sem0, sem1):
        pltpu.make_async_copy(m_hbm.at[iall0_v.at[pl.ds(ci * K, K)]], r0_v, sem0).start()
        pltpu.make_async_copy(m_hbm.at[iall1_v.at[pl.ds(ci * K, K)]], r1_v, sem1).start()

    def drain(ci, r0_v, r1_v, sem0, sem1):
        pltpu.make_async_copy(m_hbm.at[iall0_v.at[pl.ds(ci * K, K)]], r0_v, sem0).wait()
        pltpu.make_async_copy(m_hbm.at[iall1_v.at[pl.ds(ci * K, K)]], r1_v, sem1).wait()

    def compute(ci, r0_v, r1_v):
        def group(gi, carry):
            j0 = gi * L
            for jj in range(L):
                acc0 = None
                acc1 = None
                for u in range(8):
                    w0 = plsc.bitcast(r0_v[j0 + jj, pl.ds(L * u, L)], jnp.uint32)
                    w1 = plsc.bitcast(r1_v[j0 + jj, pl.ds(L * u, L)], jnp.uint32)
                    a0 = plsc.bitcast(w0 << 16, jnp.float32)
                    b0 = plsc.bitcast(w0 & jnp.uint32(0xFFFF0000), jnp.float32)
                    a1 = plsc.bitcast(w1 << 16, jnp.float32)
                    b1 = plsc.bitcast(w1 & jnp.uint32(0xFFFF0000), jnp.float32)
                    t0 = (wev[u] * a0) * a1
                    t1 = (wov[u] * b0) * b1
                    acc0 = t0 if acc0 is None else acc0 + t0
                    acc1 = t1 if acc1 is None else acc1 + t1
                p0_v[jj, :] = acc0
                p1_v[jj, :] = acc1
            o0 = jnp.zeros((L,), jnp.float32)
            o1 = jnp.zeros((L,), jnp.float32)
            for l in range(L):
                col = jnp.full((L,), l, jnp.int32)
                o0 = o0 + plsc.load_gather(p0_v, [lanes, col])
                o1 = o1 + plsc.load_gather(p1_v, [lanes, col])
            i0reg = iall0_v[pl.ds(ci * K + j0, L)]
            i1reg = iall1_v[pl.ds(ci * K + j0, L)]
            o0 = o0 + plsc.load_gather(aux0_v, [i0reg]) + plsc.load_gather(aux0_v, [i1reg])
            o1 = o1 + plsc.load_gather(aux1_v, [i0reg]) + plsc.load_gather(aux1_v, [i1reg])
            s0 = o0 + c0v
            s1 = o1 + c1v
            e2 = jnp.exp(s1 + s1)
            th = 1.0 - 2.0 / (e2 + 1.0)
            s0_v[pl.ds(j0, L)] = s0 * th * escv + biasv
            return carry

        lax.fori_loop(0, K // L, group, 0)
        base = w_base + ci * K
        pltpu.sync_copy(s0_v, out_hbm.at[pl.ds(base, K)])

    bufs = [
        (ra0_v, ra1_v, semA0, semA1),
        (rb0_v, rb1_v, semB0, semB1),
        (rc0_v, rc1_v, semC0, semC1),
        (rd0_v, rd1_v, semD0, semD1),
    ]
    depth = len(bufs)
    for t in range(depth):
        fire(t, *bufs[t])

    def quad(qi, carry):
        c = depth * qi
        for t in range(depth):
            drain(c + t, *bufs[t])
            compute(c + t, bufs[t][0], bufs[t][1])
            fire(jnp.minimum(c + t + depth, last), *bufs[t])
        return carry

    lax.fori_loop(0, nchunk // depth, quad, 0)
    for t in range(depth):
        drain(last, *bufs[t])


def _sc_contract(m, aux, i0, i1, params, w3t):
    return pl.kernel(
        _sc_body,
        out_type=jax.ShapeDtypeStruct((BQ,), jnp.float32),
        mesh=plsc.VectorSubcoreMesh(core_axis_name="c", subcore_axis_name="s"),
        compiler_params=pltpu.CompilerParams(needs_layout_passes=False),
        scratch_types=[
            pltpu.VMEM((SUMD,), jnp.float32),
            pltpu.VMEM((SUMD,), jnp.float32),
            pltpu.VMEM((4, L), jnp.float32),
            pltpu.VMEM((2, H), jnp.float32),
            pltpu.VMEM((BQ // NW,), jnp.int32),
            pltpu.VMEM((BQ // NW,), jnp.int32),
            pltpu.VMEM((K, H), jnp.int32),
            pltpu.VMEM((K, H), jnp.int32),
            pltpu.VMEM((K, H), jnp.int32),
            pltpu.VMEM((K, H), jnp.int32),
            pltpu.VMEM((K, H), jnp.int32),
            pltpu.VMEM((K, H), jnp.int32),
            pltpu.VMEM((K, H), jnp.int32),
            pltpu.VMEM((K, H), jnp.int32),
            pltpu.VMEM((K,), jnp.float32),
            pltpu.VMEM((L, L), jnp.float32),
            pltpu.VMEM((L, L), jnp.float32),
            pltpu.SemaphoreType.DMA,
            pltpu.SemaphoreType.DMA,
            pltpu.SemaphoreType.DMA,
            pltpu.SemaphoreType.DMA,
            pltpu.SemaphoreType.DMA,
            pltpu.SemaphoreType.DMA,
            pltpu.SemaphoreType.DMA,
            pltpu.SemaphoreType.DMA,
        ],
    )(m, aux, i0, i1, params, w3t)


def kernel(idxs, values, feats, candidates, scale, bias, which_axis):
    i0 = idxs[0].astype(jnp.int32)
    i1 = idxs[1].astype(jnp.int32) + D0
    m, aux, params, w3t = _prep(feats, candidates, values, scale, bias)
    return _sc_contract(m, aux, i0, i1, params, w3t)


# speedup vs baseline: 1.0218x; 1.0218x over previous
import jax
import jax.numpy as jnp
from jax import lax
from jax.experimental import pallas as pl
from jax.experimental.pallas import tpu as pltpu
from jax.experimental.pallas import tpu_sc as plsc

D0 = 4096
SUMD = 2 * D0
F = 256
H = 128
NCAND = 16
BQ = 32768
L = 16
NC, NS = 2, 16
NW = NC * NS
K = 64
RB = 1024


def _prep_body(feats_ref, cand_ref, values_ref, scale_ref, bias_ref,
               m_ref, aux_ref, params_ref, w3t_ref):
    i = pl.program_id(0)
    is0 = i < (pl.num_programs(0) // 2)
    f = feats_ref[...]
    cand = cand_ref[0]
    vals = values_ref[0]
    c0v = cand[:, 0]
    best = jnp.abs(f - c0v)
    q = jnp.broadcast_to(c0v, f.shape)
    for k in range(1, NCAND):
        ck = cand[:, k]
        d = jnp.abs(f - ck)
        better = d < best
        q = jnp.where(better, ck, q)
        best = jnp.where(better, d, best)
    w1 = vals[1] - vals[3]
    w2 = vals[2] - vals[3]
    w3 = vals[0] - vals[1] - vals[2] + vals[3]
    wa = jnp.where(is0, w1, w2)
    a0 = jnp.sum(q[:, :H] * wa[:H], axis=1)
    a1 = jnp.sum(q[:, H:] * wa[H:], axis=1)
    u = lax.bitcast_convert_type(q, jnp.uint32)
    bits = (u + jnp.uint32(0x7FFF) + ((u >> 16) & jnp.uint32(1))) >> 16
    m_ref[...] = lax.bitcast_convert_type(
        bits[:, :H] | (bits[:, H:] << 16), jnp.int32)
    aux_ref[...] = jnp.stack([a0, a1], axis=0)
    w3t_ref[...] = jnp.stack([w3[:H], w3[H:]], axis=0)
    base = vals[3]
    c0 = jnp.sum(base[:H])
    c1 = jnp.sum(base[H:])
    esc = jnp.exp(scale_ref[0])
    b = bias_ref[0]
    params_ref[...] = jnp.concatenate([
        jnp.full((1, L), esc, jnp.float32),
        jnp.full((1, L), b, jnp.float32),
        jnp.full((1, L), c0, jnp.float32),
        jnp.full((1, L), c1, jnp.float32),
    ], axis=0)


def _prep(feats, candidates, values, scale, bias):
    grid = SUMD // RB
    return pl.pallas_call(
        _prep_body,
        grid=(grid,),
        in_specs=[
            pl.BlockSpec((RB, F), lambda i: (i, 0)),
            pl.BlockSpec((1, F, NCAND), lambda i: (i // (SUMD // RB // 2), 0, 0)),
            pl.BlockSpec((1, 4, F), lambda i: (0, 0, 0)),
            pl.BlockSpec(memory_space=pltpu.SMEM),
            pl.BlockSpec(memory_space=pltpu.SMEM),
        ],
        out_specs=[
            pl.BlockSpec((RB, H), lambda i: (i, 0)),
            pl.BlockSpec((2, RB), lambda i: (0, i)),
            pl.BlockSpec((4, L), lambda i: (0, 0)),
            pl.BlockSpec((2, H), lambda i: (0, 0)),
        ],
        out_shape=[
            jax.ShapeDtypeStruct((SUMD, H), jnp.int32),
            jax.ShapeDtypeStruct((2, SUMD), jnp.float32),
            jax.ShapeDtypeStruct((4, L), jnp.float32),
            jax.ShapeDtypeStruct((2, H), jnp.float32),
        ],
    )(feats, candidates, values, scale, bias)


def _sc_body(m_hbm, aux_hbm, i0_hbm, i1_hbm, params_hbm, w3t_hbm, out_hbm,
             aux0_v, aux1_v, params_v, w3_v,
             iall0_v, iall1_v,
             ra0_v, ra1_v, rb0_v, rb1_v,
             m_sh, s0_v, p0_v, p1_v,
             semA0, semA1, semB0, semB1):
    wid = lax.axis_index("s") * NC + lax.axis_index("c")
    per_w = BQ // NW
    nchunk = per_w // K
    w_base = wid * per_w
    lanes = lax.broadcasted_iota(jnp.int32, (L,), 0)
    last = nchunk - 1

    pltpu.sync_copy(aux_hbm.at[0], aux0_v)
    pltpu.sync_copy(aux_hbm.at[1], aux1_v)
    pltpu.sync_copy(params_hbm, params_v)
    pltpu.sync_copy(w3t_hbm, w3_v)
    escv = params_v[0, :]
    biasv = params_v[1, :]
    c0v = params_v[2, :]
    c1v = params_v[3, :]
    wev = [w3_v[0, pl.ds(u * L, L)] for u in range(8)]
    wov = [w3_v[1, pl.ds(u * L, L)] for u in range(8)]

    pltpu.sync_copy(i0_hbm.at[pl.ds(w_base, per_w)], iall0_v)
    pltpu.sync_copy(i1_hbm.at[pl.ds(w_base, per_w)], iall1_v)

    sid = lax.axis_index("s")
    rows_per_sub = SUMD // NS
    pltpu.sync_copy(m_hbm.at[pl.ds(sid * rows_per_sub, rows_per_sub)],
                    m_sh.at[pl.ds(sid * rows_per_sub, rows_per_sub)])
    plsc.subcore_barrier()

    def fire(ci, r0_v, r1_v, sem0, sem1):
        pltpu.make_async_copy(m_sh.at[iall0_v.at[pl.ds(ci * K, K)]], r0_v, sem0).start()
        pltpu.make_async_copy(m_sh.at[iall1_v.at[pl.ds(ci * K, K)]], r1_v, sem1).start()

    def drain(ci, r0_v, r1_v, sem0, sem1):
        pltpu.make_async_copy(m_sh.at[iall0_v.at[pl.ds(ci * K, K)]], r0_v, sem0).wait()
        pltpu.make_async_copy(m_sh.at[iall1_v.at[pl.ds(ci * K, K)]], r1_v, sem1).wait()

    def compute(ci, r0_v, r1_v):
        def group(gi, carry):
            j0 = gi * L
            for jj in range(L):
                acc0 = None
                acc1 = None
                for u in range(8):
                    w0 = plsc.bitcast(r0_v[j0 + jj, pl.ds(L * u, L)], jnp.uint32)
                    w1 = plsc.bitcast(r1_v[j0 + jj, pl.ds(L * u, L)], jnp.uint32)
                    a0 = plsc.bitcast(w0 << 16, jnp.float32)
                    b0 = plsc.bitcast(w0 & jnp.uint32(0xFFFF0000), jnp.float32)
                    a1 = plsc.bitcast(w1 << 16, jnp.float32)
                    b1 = plsc.bitcast(w1 & jnp.uint32(0xFFFF0000), jnp.float32)
                    t0 = (wev[u] * a0) * a1
                    t1 = (wov[u] * b0) * b1
                    acc0 = t0 if acc0 is None else acc0 + t0
                    acc1 = t1 if acc1 is None else acc1 + t1
                p0_v[jj, :] = acc0
                p1_v[jj, :] = acc1
            o0 = jnp.zeros((L,), jnp.float32)
            o1 = jnp.zeros((L,), jnp.float32)
            for l in range(L):
                col = jnp.full((L,), l, jnp.int32)
                o0 = o0 + plsc.load_gather(p0_v, [lanes, col])
                o1 = o1 + plsc.load_gather(p1_v, [lanes, col])
            i0reg = iall0_v[pl.ds(ci * K + j0, L)]
            i1reg = iall1_v[pl.ds(ci * K + j0, L)]
            o0 = o0 + plsc.load_gather(aux0_v, [i0reg]) + plsc.load_gather(aux0_v, [i1reg])
            o1 = o1 + plsc.load_gather(aux1_v, [i0reg]) + plsc.load_gather(aux1_v, [i1reg])
            s0 = o0 + c0v
            s1 = o1 + c1v
            e2 = jnp.exp(s1 + s1)
            th = 1.0 - 2.0 / (e2 + 1.0)
            s0_v[pl.ds(j0, L)] = s0 * th * escv + biasv
            return carry

        lax.fori_loop(0, K // L, group, 0)
        base = w_base + ci * K
        pltpu.sync_copy(s0_v, out_hbm.at[pl.ds(base, K)])

    bufs = [
        (ra0_v, ra1_v, semA0, semA1),
        (rb0_v, rb1_v, semB0, semB1),
    ]
    depth = len(bufs)
    for t in range(depth):
        fire(t, *bufs[t])

    def quad(qi, carry):
        c = depth * qi
        for t in range(depth):
            drain(c + t, *bufs[t])
            compute(c + t, bufs[t][0], bufs[t][1])
            fire(jnp.minimum(c + t + depth, last), *bufs[t])
        return carry

    lax.fori_loop(0, nchunk // depth, quad, 0)
    for t in range(depth):
        drain(last, *bufs[t])


def _sc_contract(m, aux, i0, i1, params, w3t):
    return pl.kernel(
        _sc_body,
        out_type=jax.ShapeDtypeStruct((BQ,), jnp.float32),
        mesh=plsc.VectorSubcoreMesh(core_axis_name="c", subcore_axis_name="s"),
        compiler_params=pltpu.CompilerParams(needs_layout_passes=False),
        scratch_types=[
            pltpu.VMEM((SUMD,), jnp.float32),
            pltpu.VMEM((SUMD,), jnp.float32),
            pltpu.VMEM((4, L), jnp.float32),
            pltpu.VMEM((2, H), jnp.float32),
            pltpu.VMEM((BQ // NW,), jnp.int32),
            pltpu.VMEM((BQ // NW,), jnp.int32),
            pltpu.VMEM((K, H), jnp.int32),
            pltpu.VMEM((K, H), jnp.int32),
            pltpu.VMEM((K, H), jnp.int32),
            pltpu.VMEM((K, H), jnp.int32),
            pltpu.VMEM_SHARED((SUMD, H), jnp.int32),
            pltpu.VMEM((K,), jnp.float32),
            pltpu.VMEM((L, L), jnp.float32),
            pltpu.VMEM((L, L), jnp.float32),
            pltpu.SemaphoreType.DMA,
            pltpu.SemaphoreType.DMA,
            pltpu.SemaphoreType.DMA,
            pltpu.SemaphoreType.DMA,
        ],
    )(m, aux, i0, i1, params, w3t)


def kernel(idxs, values, feats, candidates, scale, bias, which_axis):
    i0 = idxs[0].astype(jnp.int32)
    i1 = idxs[1].astype(jnp.int32) + D0
    m, aux, params, w3t = _prep(feats, candidates, values, scale, bias)
    return _sc_contract(m, aux, i0, i1, params, w3t)


# speedup vs baseline: 1.0524x; 1.0300x over previous
import jax
import jax.numpy as jnp
from jax import lax
from jax.experimental import pallas as pl
from jax.experimental.pallas import tpu as pltpu
from jax.experimental.pallas import tpu_sc as plsc

D0 = 4096
SUMD = 2 * D0
F = 256
H = 128
NCAND = 16
BQ = 32768
L = 16
NC, NS = 2, 16
NW = NC * NS
K = 64
RB = 1024


def _prep_body(feats_ref, cand_ref, values_ref, scale_ref, bias_ref,
               m_ref, aux_ref, params_ref, w3t_ref):
    i = pl.program_id(0)
    is0 = i < (pl.num_programs(0) // 2)
    f = feats_ref[...]
    cand = cand_ref[0]
    vals = values_ref[0]
    c0v = cand[:, 0]
    best = jnp.abs(f - c0v)
    q = jnp.broadcast_to(c0v, f.shape)
    for k in range(1, NCAND):
        ck = cand[:, k]
        d = jnp.abs(f - ck)
        better = d < best
        q = jnp.where(better, ck, q)
        best = jnp.where(better, d, best)
    w1 = vals[1] - vals[3]
    w2 = vals[2] - vals[3]
    w3 = vals[0] - vals[1] - vals[2] + vals[3]
    wa = jnp.where(is0, w1, w2)
    a0 = jnp.sum(q[:, :H] * wa[:H], axis=1)
    a1 = jnp.sum(q[:, H:] * wa[H:], axis=1)
    u = lax.bitcast_convert_type(q, jnp.uint32)
    bits = (u + jnp.uint32(0x7FFF) + ((u >> 16) & jnp.uint32(1))) >> 16
    m_ref[...] = lax.bitcast_convert_type(
        bits[:, :H] | (bits[:, H:] << 16), jnp.int32)
    aux_ref[...] = jnp.stack([a0, a1], axis=0)
    w3t_ref[...] = jnp.stack([w3[:H], w3[H:]], axis=0)
    base = vals[3]
    c0 = jnp.sum(base[:H])
    c1 = jnp.sum(base[H:])
    esc = jnp.exp(scale_ref[0])
    b = bias_ref[0]
    params_ref[...] = jnp.concatenate([
        jnp.full((1, L), esc, jnp.float32),
        jnp.full((1, L), b, jnp.float32),
        jnp.full((1, L), c0, jnp.float32),
        jnp.full((1, L), c1, jnp.float32),
    ], axis=0)


def _prep(feats, candidates, values, scale, bias):
    grid = SUMD // RB
    return pl.pallas_call(
        _prep_body,
        grid=(grid,),
        in_specs=[
            pl.BlockSpec((RB, F), lambda i: (i, 0)),
            pl.BlockSpec((1, F, NCAND), lambda i: (i // (SUMD // RB // 2), 0, 0)),
            pl.BlockSpec((1, 4, F), lambda i: (0, 0, 0)),
            pl.BlockSpec(memory_space=pltpu.SMEM),
            pl.BlockSpec(memory_space=pltpu.SMEM),
        ],
        out_specs=[
            pl.BlockSpec((RB, H), lambda i: (i, 0)),
            pl.BlockSpec((2, RB), lambda i: (0, i)),
            pl.BlockSpec((4, L), lambda i: (0, 0)),
            pl.BlockSpec((2, H), lambda i: (0, 0)),
        ],
        out_shape=[
            jax.ShapeDtypeStruct((SUMD, H), jnp.int32),
            jax.ShapeDtypeStruct((2, SUMD), jnp.float32),
            jax.ShapeDtypeStruct((4, L), jnp.float32),
            jax.ShapeDtypeStruct((2, H), jnp.float32),
        ],
    )(feats, candidates, values, scale, bias)


def _sc_body(m_hbm, aux_hbm, i0_hbm, i1_hbm, params_hbm, w3t_hbm, out_hbm,
             aux0_v, aux1_v, params_v, w3_v,
             iall0_v, iall1_v,
             ra0_v, ra1_v, rb0_v, rb1_v,
             s0_v, p0_v, p1_v,
             semA0, semA1, semB0, semB1):
    wid = lax.axis_index("s") * NC + lax.axis_index("c")
    per_w = BQ // NW
    nchunk = per_w // K
    w_base = wid * per_w
    lanes = lax.broadcasted_iota(jnp.int32, (L,), 0)
    last = nchunk - 1

    pltpu.sync_copy(aux_hbm.at[0], aux0_v)
    pltpu.sync_copy(aux_hbm.at[1], aux1_v)
    pltpu.sync_copy(params_hbm, params_v)
    pltpu.sync_copy(w3t_hbm, w3_v)
    escv = params_v[0, :]
    biasv = params_v[1, :]
    c0v = params_v[2, :]
    c1v = params_v[3, :]
    wev = [w3_v[0, pl.ds(u * L, L)] for u in range(8)]
    wov = [w3_v[1, pl.ds(u * L, L)] for u in range(8)]

    pltpu.sync_copy(i0_hbm.at[pl.ds(w_base, per_w)], iall0_v)
    pltpu.sync_copy(i1_hbm.at[pl.ds(w_base, per_w)], iall1_v)

    def fire(ci, r0_v, r1_v, sem0, sem1):
        pltpu.make_async_copy(m_hbm.at[iall0_v.at[pl.ds(ci * K, K)]], r0_v, sem0).start()
        pltpu.make_async_copy(m_hbm.at[iall1_v.at[pl.ds(ci * K, K)]], r1_v, sem1).start()

    def drain(ci, r0_v, r1_v, sem0, sem1):
        pltpu.make_async_copy(m_hbm.at[iall0_v.at[pl.ds(ci * K, K)]], r0_v, sem0).wait()
        pltpu.make_async_copy(m_hbm.at[iall1_v.at[pl.ds(ci * K, K)]], r1_v, sem1).wait()

    def compute(ci, r0_v, r1_v):
        def group(gi, carry):
            j0 = gi * L
            for jj in range(L):
                acc0 = None
                acc1 = None
                for u in range(8):
                    w0 = plsc.bitcast(r0_v[j0 + jj, pl.ds(L * u, L)], jnp.uint32)
                    w1 = plsc.bitcast(r1_v[j0 + jj, pl.ds(L * u, L)], jnp.uint32)
                    a0 = plsc.bitcast(w0 << 16, jnp.float32)
                    b0 = plsc.bitcast(w0 & jnp.uint32(0xFFFF0000), jnp.float32)
                    a1 = plsc.bitcast(w1 << 16, jnp.float32)
                    b1 = plsc.bitcast(w1 & jnp.uint32(0xFFFF0000), jnp.float32)
                    t0 = (wev[u] * a0) * a1
                    t1 = (wov[u] * b0) * b1
                    acc0 = t0 if acc0 is None else acc0 + t0
                    acc1 = t1 if acc1 is None else acc1 + t1
                p0_v[jj, :] = acc0
                p1_v[jj, :] = acc1
            o0 = jnp.zeros((L,), jnp.float32)
            o1 = jnp.zeros((L,), jnp.float32)
            for l in range(L):
                col = jnp.full((L,), l, jnp.int32)
                o0 = o0 + plsc.load_gather(p0_v, [lanes, col])
                o1 = o1 + plsc.load_gather(p1_v, [lanes, col])
            i0reg = iall0_v[pl.ds(ci * K + j0, L)]
            i1reg = iall1_v[pl.ds(ci * K + j0, L)]
            o0 = o0 + plsc.load_gather(aux0_v, [i0reg]) + plsc.load_gather(aux0_v, [i1reg])
            o1 = o1 + plsc.load_gather(aux1_v, [i0reg]) + plsc.load_gather(aux1_v, [i1reg])
            s0 = o0 + c0v
            s1 = o1 + c1v
            e2 = jnp.exp(s1 + s1)
            th = 1.0 - 2.0 / (e2 + 1.0)
            s0_v[pl.ds(j0, L)] = s0 * th * escv + biasv
            return carry

        lax.fori_loop(0, K // L, group, 0)
        base = w_base + ci * K
        pltpu.sync_copy(s0_v, out_hbm.at[pl.ds(base, K)])

    bufs = [
        (ra0_v, ra1_v, semA0, semA1),
        (rb0_v, rb1_v, semB0, semB1),
    ]
    depth = len(bufs)
    for t in range(depth):
        fire(t, *bufs[t])

    def quad(qi, carry):
        c = depth * qi
        for t in range(depth):
            drain(c + t, *bufs[t])
            compute(c + t, bufs[t][0], bufs[t][1])
            fire(jnp.minimum(c + t + depth, last), *bufs[t])
        return carry

    lax.fori_loop(0, nchunk // depth, quad, 0)
    for t in range(depth):
        drain(last, *bufs[t])


def _sc_contract(m, aux, i0, i1, params, w3t):
    return pl.kernel(
        _sc_body,
        out_type=jax.ShapeDtypeStruct((BQ,), jnp.float32),
        mesh=plsc.VectorSubcoreMesh(core_axis_name="c", subcore_axis_name="s"),
        compiler_params=pltpu.CompilerParams(needs_layout_passes=False),
        scratch_types=[
            pltpu.VMEM((SUMD,), jnp.float32),
            pltpu.VMEM((SUMD,), jnp.float32),
            pltpu.VMEM((4, L), jnp.float32),
            pltpu.VMEM((2, H), jnp.float32),
            pltpu.VMEM((BQ // NW,), jnp.int32),
            pltpu.VMEM((BQ // NW,), jnp.int32),
            pltpu.VMEM((K, H), jnp.int32),
            pltpu.VMEM((K, H), jnp.int32),
            pltpu.VMEM((K, H), jnp.int32),
            pltpu.VMEM((K, H), jnp.int32),
            pltpu.VMEM((K,), jnp.float32),
            pltpu.VMEM((L, L), jnp.float32),
            pltpu.VMEM((L, L), jnp.float32),
            pltpu.SemaphoreType.DMA,
            pltpu.SemaphoreType.DMA,
            pltpu.SemaphoreType.DMA,
            pltpu.SemaphoreType.DMA,
        ],
    )(m, aux, i0, i1, params, w3t)


def kernel(idxs, values, feats, candidates, scale, bias, which_axis):
    i0 = idxs[0].astype(jnp.int32)
    i1 = idxs[1].astype(jnp.int32) + D0
    m, aux, params, w3t = _prep(feats, candidates, values, scale, bias)
    return _sc_contract(m, aux, i0, i1, params, w3t)


# speedup vs baseline: 1.0585x; 1.0058x over previous
import jax
import jax.numpy as jnp
from jax import lax
from jax.experimental import pallas as pl
from jax.experimental.pallas import tpu as pltpu
from jax.experimental.pallas import tpu_sc as plsc

D0 = 4096
SUMD = 2 * D0
F = 256
H = 128
NCAND = 16
BQ = 32768
L = 16
NC, NS = 2, 16
NW = NC * NS
K = 64
RB = 1024


def _prep_body(feats_ref, cand_ref, values_ref, scale_ref, bias_ref,
               m_ref, aux_ref, params_ref, w3t_ref):
    i = pl.program_id(0)
    is0 = i < (pl.num_programs(0) // 2)
    f = feats_ref[...]
    cand = cand_ref[0]
    vals = values_ref[0]
    c0v = cand[:, 0]
    best = jnp.abs(f - c0v)
    q = jnp.broadcast_to(c0v, f.shape)
    for k in range(1, NCAND):
        ck = cand[:, k]
        d = jnp.abs(f - ck)
        better = d < best
        q = jnp.where(better, ck, q)
        best = jnp.where(better, d, best)
    w1 = vals[1] - vals[3]
    w2 = vals[2] - vals[3]
    w3 = vals[0] - vals[1] - vals[2] + vals[3]
    wa = jnp.where(is0, w1, w2)
    a0 = jnp.sum(q[:, :H] * wa[:H], axis=1)
    a1 = jnp.sum(q[:, H:] * wa[H:], axis=1)
    u = lax.bitcast_convert_type(q, jnp.uint32)
    bits = (u + jnp.uint32(0x7FFF) + ((u >> 16) & jnp.uint32(1))) >> 16
    m_ref[...] = lax.bitcast_convert_type(
        bits[:, :H] | (bits[:, H:] << 16), jnp.int32)
    aux_ref[...] = jnp.stack([a0, a1], axis=0)
    w3t_ref[...] = jnp.stack([w3[:H], w3[H:]], axis=0)
    base = vals[3]
    c0 = jnp.sum(base[:H])
    c1 = jnp.sum(base[H:])
    esc = jnp.exp(scale_ref[0])
    b = bias_ref[0]
    params_ref[...] = jnp.concatenate([
        jnp.full((1, L), esc, jnp.float32),
        jnp.full((1, L), b, jnp.float32),
        jnp.full((1, L), c0, jnp.float32),
        jnp.full((1, L), c1, jnp.float32),
    ], axis=0)


def _prep(feats, candidates, values, scale, bias):
    grid = SUMD // RB
    return pl.pallas_call(
        _prep_body,
        grid=(grid,),
        in_specs=[
            pl.BlockSpec((RB, F), lambda i: (i, 0)),
            pl.BlockSpec((1, F, NCAND), lambda i: (i // (SUMD // RB // 2), 0, 0)),
            pl.BlockSpec((1, 4, F), lambda i: (0, 0, 0)),
            pl.BlockSpec(memory_space=pltpu.SMEM),
            pl.BlockSpec(memory_space=pltpu.SMEM),
        ],
        out_specs=[
            pl.BlockSpec((RB, H), lambda i: (i, 0)),
            pl.BlockSpec((2, RB), lambda i: (0, i)),
            pl.BlockSpec((4, L), lambda i: (0, 0)),
            pl.BlockSpec((2, H), lambda i: (0, 0)),
        ],
        out_shape=[
            jax.ShapeDtypeStruct((SUMD, H), jnp.int32),
            jax.ShapeDtypeStruct((2, SUMD), jnp.float32),
            jax.ShapeDtypeStruct((4, L), jnp.float32),
            jax.ShapeDtypeStruct((2, H), jnp.float32),
        ],
    )(feats, candidates, values, scale, bias)


def _sc_body(m_hbm, aux_hbm, i0_hbm, i1_hbm, params_hbm, w3t_hbm, out_hbm,
             aux0_v, aux1_v, params_v, w3_v,
             iall0_v, iall1_v,
             ra0_v, ra1_v, rb0_v, rb1_v,
             s0_v, p0_v, p1_v,
             semA0, semA1, semB0, semB1):
    wid = lax.axis_index("s") * NC + lax.axis_index("c")
    per_w = BQ // NW
    nchunk = per_w // K
    w_base = wid * per_w
    lanes = lax.broadcasted_iota(jnp.int32, (L,), 0)
    last = nchunk - 1

    pltpu.sync_copy(aux_hbm.at[0], aux0_v)
    pltpu.sync_copy(aux_hbm.at[1], aux1_v)
    pltpu.sync_copy(params_hbm, params_v)
    pltpu.sync_copy(w3t_hbm, w3_v)
    escv = params_v[0, :]
    biasv = params_v[1, :]
    c0v = params_v[2, :]
    c1v = params_v[3, :]
    wev = [w3_v[0, pl.ds(u * L, L)] for u in range(8)]
    wov = [w3_v[1, pl.ds(u * L, L)] for u in range(8)]

    pltpu.sync_copy(i0_hbm.at[pl.ds(w_base, per_w)], iall0_v)
    pltpu.sync_copy(i1_hbm.at[pl.ds(w_base, per_w)], iall1_v)

    def fire(ci, r0_v, r1_v, sem0, sem1):
        pltpu.make_async_copy(m_hbm.at[iall0_v.at[pl.ds(ci * K, K)]], r0_v, sem0).start()
        pltpu.make_async_copy(m_hbm.at[iall1_v.at[pl.ds(ci * K, K)]], r1_v, sem1).start()

    def drain(ci, r0_v, r1_v, sem0, sem1):
        pltpu.make_async_copy(m_hbm.at[iall0_v.at[pl.ds(ci * K, K)]], r0_v, sem0).wait()
        pltpu.make_async_copy(m_hbm.at[iall1_v.at[pl.ds(ci * K, K)]], r1_v, sem1).wait()

    def compute(ci, r0_v, r1_v):
        def group(gi, carry):
            j0 = gi * L
            for jj in range(L):
                acc0 = None
                acc1 = None
                for u in range(8):
                    w0 = plsc.bitcast(r0_v[j0 + jj, pl.ds(L * u, L)], jnp.uint32)
                    w1 = plsc.bitcast(r1_v[j0 + jj, pl.ds(L * u, L)], jnp.uint32)
                    a0 = plsc.bitcast(w0 << 16, jnp.float32)
                    b0 = plsc.bitcast(w0 & jnp.uint32(0xFFFF0000), jnp.float32)
                    a1 = plsc.bitcast(w1 << 16, jnp.float32)
                    b1 = plsc.bitcast(w1 & jnp.uint32(0xFFFF0000), jnp.float32)
                    t0 = (wev[u] * a0) * a1
                    t1 = (wov[u] * b0) * b1
                    acc0 = t0 if acc0 is None else acc0 + t0
                    acc1 = t1 if acc1 is None else acc1 + t1
                p0_v[jj, :] = acc0
                p1_v[jj, :] = acc1
            o0 = jnp.zeros((L,), jnp.float32)
            o1 = jnp.zeros((L,), jnp.float32)
            for l in range(L):
                col = jnp.full((L,), l, jnp.int32)
                o0 = o0 + plsc.load_gather(p0_v, [lanes, col])
                o1 = o1 + plsc.load_gather(p1_v, [lanes, col])
            i0reg = iall0_v[pl.ds(ci * K + j0, L)]
            i1reg = iall1_v[pl.ds(ci * K + j0, L)]
            o0 = o0 + plsc.load_gather(aux0_v, [i0reg]) + plsc.load_gather(aux0_v, [i1reg])
            o1 = o1 + plsc.load_gather(aux1_v, [i0reg]) + plsc.load_gather(aux1_v, [i1reg])
            s0 = o0 + c0v
            s1 = o1 + c1v
            e2 = jnp.exp(s1 + s1)
            th = 1.0 - 2.0 / (e2 + 1.0)
            s0_v[pl.ds(ci * K + j0, L)] = s0 * th * escv + biasv
            return carry

        lax.fori_loop(0, K // L, group, 0)

    bufs = [
        (ra0_v, ra1_v, semA0, semA1),
        (rb0_v, rb1_v, semB0, semB1),
    ]
    depth = len(bufs)
    for t in range(depth):
        fire(t, *bufs[t])

    def quad(qi, carry):
        c = depth * qi
        for t in range(depth):
            drain(c + t, *bufs[t])
            compute(c + t, bufs[t][0], bufs[t][1])
            fire(jnp.minimum(c + t + depth, last), *bufs[t])
        return carry

    lax.fori_loop(0, nchunk // depth, quad, 0)
    for t in range(depth):
        drain(last, *bufs[t])
    pltpu.sync_copy(s0_v, out_hbm.at[pl.ds(w_base, per_w)])


def _sc_contract(m, aux, i0, i1, params, w3t):
    return pl.kernel(
        _sc_body,
        out_type=jax.ShapeDtypeStruct((BQ,), jnp.float32),
        mesh=plsc.VectorSubcoreMesh(core_axis_name="c", subcore_axis_name="s"),
        compiler_params=pltpu.CompilerParams(needs_layout_passes=False),
        scratch_types=[
            pltpu.VMEM((SUMD,), jnp.float32),
            pltpu.VMEM((SUMD,), jnp.float32),
            pltpu.VMEM((4, L), jnp.float32),
            pltpu.VMEM((2, H), jnp.float32),
            pltpu.VMEM((BQ // NW,), jnp.int32),
            pltpu.VMEM((BQ // NW,), jnp.int32),
            pltpu.VMEM((K, H), jnp.int32),
            pltpu.VMEM((K, H), jnp.int32),
            pltpu.VMEM((K, H), jnp.int32),
            pltpu.VMEM((K, H), jnp.int32),
            pltpu.VMEM((BQ // NW,), jnp.float32),
            pltpu.VMEM((L, L), jnp.float32),
            pltpu.VMEM((L, L), jnp.float32),
            pltpu.SemaphoreType.DMA,
            pltpu.SemaphoreType.DMA,
            pltpu.SemaphoreType.DMA,
            pltpu.SemaphoreType.DMA,
        ],
    )(m, aux, i0, i1, params, w3t)


def kernel(idxs, values, feats, candidates, scale, bias, which_axis):
    i0 = idxs[0].astype(jnp.int32)
    i1 = idxs[1].astype(jnp.int32) + D0
    m, aux, params, w3t = _prep(feats, candidates, values, scale, bias)
    return _sc_contract(m, aux, i0, i1, params, w3t)
